# baseline (device time: 23832 ns/iter reference)
import jax
import jax.numpy as jnp
from jax import lax
from jax.experimental import pallas as pl
from jax.experimental.pallas import tpu as pltpu

N_DEV = 4
B, H, D, BS = 16, 16, 64, 16
P = 512 // N_DEV
NT = 128


def kernel(Q, K, V, bt, lens):
    lens2 = lens.reshape(B, 1)
    Kp = jnp.transpose(K, (1, 2, 3, 0))
    Vp = jnp.transpose(V, (1, 2, 3, 0))

    def body(q_ref, k_ref, v_ref, bt_ref, lens_ref, out_ref,
             comm_ref, send_sems, recv_sems, k_vmem, v_vmem, kv_sems):
        my = lax.axis_index("i")

        cpk = pltpu.make_async_copy(k_ref, k_vmem, kv_sems.at[0])
        cpv = pltpu.make_async_copy(v_ref, v_vmem, kv_sems.at[1])
        cpk.start()
        cpv.start()

        barrier_sem = pltpu.get_barrier_semaphore()
        for t in range(1, N_DEV):
            pl.semaphore_signal(barrier_sem, inc=1,
                                device_id=(lax.rem(my + t, N_DEV),),
                                device_id_type=pl.DeviceIdType.MESH)
        pl.semaphore_wait(barrier_sem, N_DEV - 1)

        page0 = my * P
        btv = bt_ref[:, :]
        lensv = lens_ref[:, :]
        g = page0 + lax.broadcasted_iota(jnp.int32, (1, 1, P), 2)
        j = lax.broadcasted_iota(jnp.int32, (1, NT, 1), 1)
        match = (btv[:, :, None] == g) & (j < lensv[:, :, None])
        cnt = jnp.sum(match.astype(jnp.float32), axis=1)
        logcnt = jnp.log(cnt)

        scale = D ** -0.5
        q = q_ref[:, 0, :, :].transpose(1, 0, 2)
        qb = jnp.broadcast_to(q[None], (BS, H, B, D)).reshape(BS * H, B, D)
        cpk.wait()
        k = k_vmem[:, :, :, :].reshape(BS * H, D, P)
        s = jax.lax.dot_general(
            qb, k, (((2,), (1,)), ((0,), (0,))),
            preferred_element_type=jnp.float32) * scale
        s4 = s.reshape(BS, H, B, P) + logcnt[None, None, :, :]
        m = jnp.maximum(
            jnp.max(s4, axis=(0, 3), keepdims=True), -1e30)
        e4 = jnp.exp(s4 - m)
        l = jnp.sum(e4, axis=(0, 3), keepdims=True)
        cpv.wait()
        v = v_vmem[:, :, :, :].reshape(BS * H, D, P)
        o = jax.lax.dot_general(
            e4.reshape(BS * H, B, P), v, (((2,), (2,)), ((0,), (0,))),
            preferred_element_type=jnp.float32)
        o = jnp.sum(o.reshape(BS, H, B, D), axis=0)
        comm_ref[0, :, 0:D] = o.reshape(H * B, D)
        comm_ref[0, :, D:D + 1] = m.reshape(H * B, 1)
        comm_ref[0, :, D + 1:D + 2] = l.reshape(H * B, 1)

        rdmas = []
        for t in range(1, N_DEV):
            rdma = pltpu.make_async_remote_copy(
                src_ref=comm_ref.at[0],
                dst_ref=comm_ref.at[N_DEV - t],
                send_sem=send_sems.at[N_DEV - 1 - t],
                recv_sem=recv_sems.at[N_DEV - 1 - t],
                device_id=(lax.rem(my + t, N_DEV),),
                device_id_type=pl.DeviceIdType.MESH,
            )
            rdma.start()
            rdmas.append(rdma)
        for rdma in rdmas:
            rdma.wait_send()
        for rdma in rdmas:
            rdma.wait_recv()

        os_ = comm_ref[:, :, 0:D]
        ms = comm_ref[:, :, D:D + 1]
        ls = comm_ref[:, :, D + 1:D + 2]
        mg = jnp.max(ms, axis=0)
        sc = jnp.exp(ms - mg[None, :, :])
        lg = jnp.sum(ls * sc, axis=0)
        og = jnp.sum(os_ * sc, axis=0)
        out = og / lg
        out_ref[:, :, :, :] = (
            out.reshape(H, B, D).transpose(1, 0, 2).reshape(B, 1, H, D)
        )

    return pl.pallas_call(
        body,
        out_shape=jax.ShapeDtypeStruct((B, 1, H, D), jnp.float32),
        in_specs=[
            pl.BlockSpec(memory_space=pltpu.VMEM),
            pl.BlockSpec(memory_space=pl.ANY),
            pl.BlockSpec(memory_space=pl.ANY),
            pl.BlockSpec(memory_space=pltpu.VMEM),
            pl.BlockSpec(memory_space=pltpu.VMEM),
        ],
        out_specs=pl.BlockSpec(memory_space=pltpu.VMEM),
        scratch_shapes=[
            pltpu.VMEM((N_DEV, B * H, 128), jnp.float32),
            pltpu.SemaphoreType.DMA((N_DEV - 1,)),
            pltpu.SemaphoreType.DMA((N_DEV - 1,)),
            pltpu.VMEM((BS, H, D, P), jnp.float32),
            pltpu.VMEM((BS, H, D, P), jnp.float32),
            pltpu.SemaphoreType.DMA((2,)),
        ],
        compiler_params=pltpu.CompilerParams(collective_id=0),
    )(Q, Kp, Vp, bt, lens2)


# device time: 23827 ns/iter; 1.0002x vs baseline; 1.0002x over previous
import jax
import jax.numpy as jnp
from jax import lax
from jax.experimental import pallas as pl
from jax.experimental.pallas import tpu as pltpu

N_DEV = 4
B, H, D, BS = 16, 16, 64, 16
P = 512 // N_DEV
NT = 128


def kernel(Q, K, V, bt, lens):
    lens2 = lens.reshape(B, 1)
    Kp = jnp.transpose(K, (1, 2, 3, 0))
    Vp = jnp.transpose(V, (1, 2, 3, 0))

    def body(q_ref, k_ref, v_ref, bt_ref, lens_ref, out_ref,
             comm_ref, send_sems, recv_sems, k_vmem, v_vmem, kv_sems):
        my = lax.axis_index("i")

        cpk = pltpu.make_async_copy(k_ref, k_vmem, kv_sems.at[0])
        cpv = pltpu.make_async_copy(v_ref, v_vmem, kv_sems.at[1])
        cpk.start()
        cpv.start()

        barrier_sem = pltpu.get_barrier_semaphore()
        for t in range(1, N_DEV):
            pl.semaphore_signal(barrier_sem, inc=1,
                                device_id=(lax.rem(my + t, N_DEV),),
                                device_id_type=pl.DeviceIdType.MESH)
        pl.semaphore_wait(barrier_sem, N_DEV - 1)

        page0 = my * P
        btv = bt_ref[:, :]
        lensv = lens_ref[:, :]
        g = page0 + lax.broadcasted_iota(jnp.int32, (1, 1, P), 2)
        j = lax.broadcasted_iota(jnp.int32, (1, NT, 1), 1)
        match = (btv[:, :, None] == g) & (j < lensv[:, :, None])
        cnt = jnp.sum(match.astype(jnp.float32), axis=1)
        logcnt = jnp.log(cnt)

        scale = D ** -0.5
        q = q_ref[:, 0, :, :].transpose(1, 0, 2)
        qb = jnp.broadcast_to(q[None], (BS, H, B, D)).reshape(BS * H, B, D)
        cpk.wait()
        k = k_vmem[:, :, :, :].reshape(BS * H, D, P)
        s = jax.lax.dot_general(
            qb, k, (((2,), (1,)), ((0,), (0,))),
            preferred_element_type=jnp.float32) * scale
        s4 = s.reshape(BS, H, B, P) + logcnt[None, None, :, :]
        m = jnp.maximum(
            jnp.max(s4, axis=(0, 3), keepdims=True), -1e30)
        e4 = jnp.exp(s4 - m)
        l = jnp.sum(e4, axis=(0, 3), keepdims=True)
        cpv.wait()
        v = v_vmem[:, :, :, :].reshape(BS * H, D, P)
        o = jax.lax.dot_general(
            e4.reshape(BS * H, B, P), v, (((2,), (2,)), ((0,), (0,))),
            preferred_element_type=jnp.float32)
        o = jnp.sum(o.reshape(BS, H, B, D), axis=0)
        comm_ref[0, :, 0:D] = o.reshape(H * B, D)
        comm_ref[0, :, D:D + 1] = m.reshape(H * B, 1)
        comm_ref[0, :, D + 1:D + 2] = l.reshape(H * B, 1)

        rdmas = []
        for t in range(1, N_DEV):
            rdma = pltpu.make_async_remote_copy(
                src_ref=comm_ref.at[0],
                dst_ref=comm_ref.at[N_DEV - t],
                send_sem=send_sems.at[N_DEV - 1 - t],
                recv_sem=recv_sems.at[N_DEV - 1 - t],
                device_id=(lax.rem(my + t, N_DEV),),
                device_id_type=pl.DeviceIdType.MESH,
            )
            rdma.start()
            rdmas.append(rdma)
        for rdma in rdmas:
            rdma.wait_send()
        for rdma in rdmas:
            rdma.wait_recv()

        os_ = comm_ref[:, :, 0:D]
        ms = comm_ref[:, :, D:D + 1]
        ls = comm_ref[:, :, D + 1:D + 2]
        mg = jnp.max(ms, axis=0)
        sc = jnp.exp(ms - mg[None, :, :])
        lg = jnp.sum(ls * sc, axis=0)
        og = jnp.sum(os_ * sc, axis=0)
        out = og / lg
        out_ref[:, :, :, :] = (
            out.reshape(H, B, D).transpose(1, 0, 2).reshape(B, 1, H, D)
        )

    return pl.pallas_call(
        body,
        out_shape=jax.ShapeDtypeStruct((B, 1, H, D), jnp.float32),
        in_specs=[
            pl.BlockSpec(memory_space=pltpu.VMEM),
            pl.BlockSpec(memory_space=pltpu.MemorySpace.HBM),
            pl.BlockSpec(memory_space=pltpu.MemorySpace.HBM),
            pl.BlockSpec(memory_space=pltpu.VMEM),
            pl.BlockSpec(memory_space=pltpu.VMEM),
        ],
        out_specs=pl.BlockSpec(memory_space=pltpu.VMEM),
        scratch_shapes=[
            pltpu.VMEM((N_DEV, B * H, 128), jnp.float32),
            pltpu.SemaphoreType.DMA((N_DEV - 1,)),
            pltpu.SemaphoreType.DMA((N_DEV - 1,)),
            pltpu.VMEM((BS, H, D, P), jnp.float32),
            pltpu.VMEM((BS, H, D, P), jnp.float32),
            pltpu.SemaphoreType.DMA((2,)),
        ],
        compiler_params=pltpu.CompilerParams(collective_id=0),
    )(Q, Kp, Vp, bt, lens2)


# device time: 22140 ns/iter; 1.0764x vs baseline; 1.0762x over previous
import jax
import jax.numpy as jnp
from jax import lax
from jax.experimental import pallas as pl
from jax.experimental.pallas import tpu as pltpu

N_DEV = 4
B, H, D, BS = 16, 16, 64, 16
P = 512 // N_DEV
NT = 128


def kernel(Q, K, V, bt, lens):
    Kp = jnp.transpose(K, (1, 2, 3, 0))
    Vp = jnp.transpose(V, (1, 2, 3, 0))
    fused = jnp.concatenate(
        [
            Q.reshape(B, H * D),
            lax.bitcast_convert_type(bt, jnp.float32),
            lax.bitcast_convert_type(lens.reshape(B, 1), jnp.float32),
        ],
        axis=1,
    )

    def body(f_ref, k_ref, v_ref, out_ref, comm_ref, send_sems, recv_sems):
        my = lax.axis_index("i")

        barrier_sem = pltpu.get_barrier_semaphore()
        for t in range(1, N_DEV):
            pl.semaphore_signal(barrier_sem, inc=1,
                                device_id=(lax.rem(my + t, N_DEV),),
                                device_id_type=pl.DeviceIdType.MESH)
        pl.semaphore_wait(barrier_sem, N_DEV - 1)

        page0 = my * P
        btv = lax.bitcast_convert_type(
            f_ref[:, H * D:H * D + NT], jnp.int32)
        lensv = lax.bitcast_convert_type(
            f_ref[:, H * D + NT:H * D + NT + 1], jnp.int32)
        g = page0 + lax.broadcasted_iota(jnp.int32, (1, 1, P), 2)
        j = lax.broadcasted_iota(jnp.int32, (1, NT, 1), 1)
        match = (btv[:, :, None] == g) & (j < lensv[:, :, None])
        cnt = jnp.sum(match.astype(jnp.float32), axis=1)
        logcnt = jnp.log(cnt)

        scale = D ** -0.5
        q = f_ref[:, 0:H * D].reshape(B, H, D).transpose(1, 0, 2)
        qb = jnp.broadcast_to(q[None], (BS, H, B, D)).reshape(BS * H, B, D)
        k = k_ref[:, :, :, :].reshape(BS * H, D, P)
        v = v_ref[:, :, :, :].reshape(BS * H, D, P)
        s = jax.lax.dot_general(
            qb, k, (((2,), (1,)), ((0,), (0,))),
            preferred_element_type=jnp.float32) * scale
        s4 = s.reshape(BS, H, B, P) + logcnt[None, None, :, :]
        m = jnp.maximum(
            jnp.max(s4, axis=(0, 3), keepdims=True), -1e30)
        e4 = jnp.exp(s4 - m)
        l = jnp.sum(e4, axis=(0, 3), keepdims=True)
        o = jax.lax.dot_general(
            e4.reshape(BS * H, B, P), v, (((2,), (2,)), ((0,), (0,))),
            preferred_element_type=jnp.float32)
        o = jnp.sum(o.reshape(BS, H, B, D), axis=0)

        comm_ref[0, :, 0:D] = o.reshape(H * B, D)
        comm_ref[0, :, D:D + 1] = m.reshape(H * B, 1)
        comm_ref[0, :, D + 1:D + 2] = l.reshape(H * B, 1)

        rdmas = []
        for t in range(1, N_DEV):
            rdma = pltpu.make_async_remote_copy(
                src_ref=comm_ref.at[0],
                dst_ref=comm_ref.at[N_DEV - t],
                send_sem=send_sems.at[N_DEV - 1 - t],
                recv_sem=recv_sems.at[N_DEV - 1 - t],
                device_id=(lax.rem(my + t, N_DEV),),
                device_id_type=pl.DeviceIdType.MESH,
            )
            rdma.start()
            rdmas.append(rdma)
        for rdma in rdmas:
            rdma.wait_send()
        for rdma in rdmas:
            rdma.wait_recv()

        os_ = comm_ref[:, :, 0:D]
        ms = comm_ref[:, :, D:D + 1]
        ls = comm_ref[:, :, D + 1:D + 2]
        mg = jnp.max(ms, axis=0)
        sc = jnp.exp(ms - mg[None, :, :])
        lg = jnp.sum(ls * sc, axis=0)
        og = jnp.sum(os_ * sc, axis=0)
        out = og / lg
        out_ref[:, :, :, :] = (
            out.reshape(H, B, D).transpose(1, 0, 2).reshape(B, 1, H, D)
        )

    return pl.pallas_call(
        body,
        out_shape=jax.ShapeDtypeStruct((B, 1, H, D), jnp.float32),
        in_specs=[pl.BlockSpec(memory_space=pltpu.VMEM)] * 3,
        out_specs=pl.BlockSpec(memory_space=pltpu.VMEM),
        scratch_shapes=[
            pltpu.VMEM((N_DEV, B * H, 128), jnp.float32),
            pltpu.SemaphoreType.DMA((N_DEV - 1,)),
            pltpu.SemaphoreType.DMA((N_DEV - 1,)),
        ],
        compiler_params=pltpu.CompilerParams(collective_id=0),
    )(fused, Kp, Vp)


# device time: 19800 ns/iter; 1.2036x vs baseline; 1.1182x over previous
import jax
import jax.numpy as jnp
from jax import lax
from jax.experimental import pallas as pl
from jax.experimental.pallas import tpu as pltpu

N_DEV = 4
B, H, D, BS = 16, 16, 64, 16
P = 512 // N_DEV
NT = 128


def kernel(Q, K, V, bt, lens):
    lens2 = lens.reshape(B, 1)
    Kp = jnp.transpose(K, (1, 2, 3, 0))
    Vp = jnp.transpose(V, (1, 2, 3, 0))

    def body(q_ref, k_ref, v_ref, bt_ref, lens_ref, out_ref,
             comm_ref, send_sems, recv_sems):
        my = lax.axis_index("i")

        barrier_sem = pltpu.get_barrier_semaphore()
        for t in range(1, N_DEV):
            pl.semaphore_signal(barrier_sem, inc=1,
                                device_id=(lax.rem(my + t, N_DEV),),
                                device_id_type=pl.DeviceIdType.MESH)

        page0 = my * P
        btv = bt_ref[:, :]
        lensv = lens_ref[:, :]
        g = page0 + lax.broadcasted_iota(jnp.int32, (1, 1, P), 2)
        j = lax.broadcasted_iota(jnp.int32, (1, NT, 1), 1)
        match = (btv[:, :, None] == g) & (j < lensv[:, :, None])
        cnt = jnp.sum(match.astype(jnp.float32), axis=1)
        logcnt = jnp.log(cnt)

        scale = D ** -0.5
        q = q_ref[:, 0, :, :].transpose(1, 0, 2)
        qb = jnp.broadcast_to(q[None], (BS, H, B, D)).reshape(BS * H, B, D)
        k = k_ref[:, :, :, :].reshape(BS * H, D, P)
        v = v_ref[:, :, :, :].reshape(BS * H, D, P)
        s = jax.lax.dot_general(
            qb, k, (((2,), (1,)), ((0,), (0,))),
            preferred_element_type=jnp.float32) * scale
        s4 = s.reshape(BS, H, B, P) + logcnt[None, None, :, :]
        m = jnp.maximum(
            jnp.max(s4, axis=(0, 3), keepdims=True), -1e30)
        e4 = jnp.exp(s4 - m)
        l = jnp.sum(e4, axis=(0, 3), keepdims=True)
        o = jax.lax.dot_general(
            e4.reshape(BS * H, B, P), v, (((2,), (2,)), ((0,), (0,))),
            preferred_element_type=jnp.float32)
        o = jnp.sum(o.reshape(BS, H, B, D), axis=0)

        mr = m.reshape(H * B, 1)
        lr = l.reshape(H * B, 1)
        m_hi = mr.astype(jnp.bfloat16)
        m_lo = (mr - m_hi.astype(jnp.float32)).astype(jnp.bfloat16)
        l_hi = lr.astype(jnp.bfloat16)
        l_lo = (lr - l_hi.astype(jnp.float32)).astype(jnp.bfloat16)
        comm_ref[0, :, 0:D] = o.reshape(H * B, D).astype(jnp.bfloat16)
        comm_ref[0, :, D:D + 1] = m_hi
        comm_ref[0, :, D + 1:D + 2] = m_lo
        comm_ref[0, :, D + 2:D + 3] = l_hi
        comm_ref[0, :, D + 3:D + 4] = l_lo

        pl.semaphore_wait(barrier_sem, N_DEV - 1)

        rdmas = []
        for t in range(1, N_DEV):
            rdma = pltpu.make_async_remote_copy(
                src_ref=comm_ref.at[0],
                dst_ref=comm_ref.at[N_DEV - t],
                send_sem=send_sems.at[N_DEV - 1 - t],
                recv_sem=recv_sems.at[N_DEV - 1 - t],
                device_id=(lax.rem(my + t, N_DEV),),
                device_id_type=pl.DeviceIdType.MESH,
            )
            rdma.start()
            rdmas.append(rdma)
        for rdma in rdmas:
            rdma.wait_send()
        for rdma in rdmas:
            rdma.wait_recv()

        os_ = comm_ref[:, :, 0:D].astype(jnp.float32)
        ms = (comm_ref[:, :, D:D + 1].astype(jnp.float32)
              + comm_ref[:, :, D + 1:D + 2].astype(jnp.float32))
        ls = (comm_ref[:, :, D + 2:D + 3].astype(jnp.float32)
              + comm_ref[:, :, D + 3:D + 4].astype(jnp.float32))
        mg = jnp.max(ms, axis=0)
        sc = jnp.exp(ms - mg[None, :, :])
        lg = jnp.sum(ls * sc, axis=0)
        og = jnp.sum(os_ * sc, axis=0)
        out = og / lg
        out_ref[:, :, :, :] = (
            out.reshape(H, B, D).transpose(1, 0, 2).reshape(B, 1, H, D)
        )

    return pl.pallas_call(
        body,
        out_shape=jax.ShapeDtypeStruct((B, 1, H, D), jnp.float32),
        in_specs=[pl.BlockSpec(memory_space=pltpu.VMEM)] * 5,
        out_specs=pl.BlockSpec(memory_space=pltpu.VMEM),
        scratch_shapes=[
            pltpu.VMEM((N_DEV, B * H, 128), jnp.bfloat16),
            pltpu.SemaphoreType.DMA((N_DEV - 1,)),
            pltpu.SemaphoreType.DMA((N_DEV - 1,)),
        ],
        compiler_params=pltpu.CompilerParams(collective_id=0),
    )(Q, Kp, Vp, bt, lens2)


# device time: 19748 ns/iter; 1.2068x vs baseline; 1.0026x over previous
import jax
import jax.numpy as jnp
from jax import lax
from jax.experimental import pallas as pl
from jax.experimental.pallas import tpu as pltpu

N_DEV = 4
B, H, D, BS = 16, 16, 64, 16
P = 512 // N_DEV
NT = 128


def kernel(Q, K, V, bt, lens):
    lens2 = lens.reshape(B, 1)
    Kp = jnp.transpose(K, (1, 2, 3, 0))
    Vp = jnp.transpose(V, (1, 2, 3, 0))

    def body(q_ref, k_ref, v_ref, bt_ref, lens_ref, out_ref,
             comm_ref, send_sems, recv_sems):
        my = lax.axis_index("i")

        barrier_sem = pltpu.get_barrier_semaphore()
        for t in range(1, N_DEV):
            pl.semaphore_signal(barrier_sem, inc=1,
                                device_id=(lax.rem(my + t, N_DEV),),
                                device_id_type=pl.DeviceIdType.MESH)

        page0 = my * P
        btv = bt_ref[:, :]
        lensv = lens_ref[:, :]
        g = page0 + lax.broadcasted_iota(jnp.int32, (1, 1, P), 2)
        j = lax.broadcasted_iota(jnp.int32, (1, NT, 1), 1)
        match = (btv[:, :, None] == g) & (j < lensv[:, :, None])
        cnt = jnp.sum(match.astype(jnp.float32), axis=1)
        logcnt = jnp.log(cnt)

        scale = D ** -0.5
        HH = H // 2
        q = q_ref[:, 0, :, :].transpose(1, 0, 2)
        rdmas = []

        for w in range(2):
            h0 = w * HH
            qh = q[h0:h0 + HH]
            qb = jnp.broadcast_to(
                qh[None], (BS, HH, B, D)).reshape(BS * HH, B, D)
            k = k_ref[:, h0:h0 + HH, :, :].reshape(BS * HH, D, P)
            v = v_ref[:, h0:h0 + HH, :, :].reshape(BS * HH, D, P)
            s = jax.lax.dot_general(
                qb, k, (((2,), (1,)), ((0,), (0,))),
                preferred_element_type=jnp.float32) * scale
            s4 = s.reshape(BS, HH, B, P) + logcnt[None, None, :, :]
            m = jnp.maximum(
                jnp.max(s4, axis=(0, 3), keepdims=True), -1e30)
            e4 = jnp.exp(s4 - m)
            l = jnp.sum(e4, axis=(0, 3), keepdims=True)
            o = jax.lax.dot_general(
                e4.reshape(BS * HH, B, P), v, (((2,), (2,)), ((0,), (0,))),
                preferred_element_type=jnp.float32)
            o = jnp.sum(o.reshape(BS, HH, B, D), axis=0)

            r0 = h0 * B
            nr = HH * B
            mr = m.reshape(nr, 1)
            lr = l.reshape(nr, 1)
            m_hi = mr.astype(jnp.bfloat16)
            m_lo = (mr - m_hi.astype(jnp.float32)).astype(jnp.bfloat16)
            l_hi = lr.astype(jnp.bfloat16)
            l_lo = (lr - l_hi.astype(jnp.float32)).astype(jnp.bfloat16)
            comm_ref[0, pl.ds(r0, nr), 0:D] = o.reshape(nr, D).astype(
                jnp.bfloat16)
            comm_ref[0, pl.ds(r0, nr), D:D + 1] = m_hi
            comm_ref[0, pl.ds(r0, nr), D + 1:D + 2] = m_lo
            comm_ref[0, pl.ds(r0, nr), D + 2:D + 3] = l_hi
            comm_ref[0, pl.ds(r0, nr), D + 3:D + 4] = l_lo

            if w == 0:
                pl.semaphore_wait(barrier_sem, N_DEV - 1)

            for t in range(1, N_DEV):
                rdma = pltpu.make_async_remote_copy(
                    src_ref=comm_ref.at[0, pl.ds(r0, nr)],
                    dst_ref=comm_ref.at[N_DEV - t, pl.ds(r0, nr)],
                    send_sem=send_sems.at[w, N_DEV - 1 - t],
                    recv_sem=recv_sems.at[w, N_DEV - 1 - t],
                    device_id=(lax.rem(my + t, N_DEV),),
                    device_id_type=pl.DeviceIdType.MESH,
                )
                rdma.start()
                rdmas.append(rdma)

        for rdma in rdmas:
            rdma.wait_send()
        for rdma in rdmas:
            rdma.wait_recv()

        os_ = comm_ref[:, :, 0:D].astype(jnp.float32)
        ms = (comm_ref[:, :, D:D + 1].astype(jnp.float32)
              + comm_ref[:, :, D + 1:D + 2].astype(jnp.float32))
        ls = (comm_ref[:, :, D + 2:D + 3].astype(jnp.float32)
              + comm_ref[:, :, D + 3:D + 4].astype(jnp.float32))
        mg = jnp.max(ms, axis=0)
        sc = jnp.exp(ms - mg[None, :, :])
        lg = jnp.sum(ls * sc, axis=0)
        og = jnp.sum(os_ * sc, axis=0)
        out = og / lg
        out_ref[:, :, :, :] = (
            out.reshape(H, B, D).transpose(1, 0, 2).reshape(B, 1, H, D)
        )

    return pl.pallas_call(
        body,
        out_shape=jax.ShapeDtypeStruct((B, 1, H, D), jnp.float32),
        in_specs=[pl.BlockSpec(memory_space=pltpu.VMEM)] * 5,
        out_specs=pl.BlockSpec(memory_space=pltpu.VMEM),
        scratch_shapes=[
            pltpu.VMEM((N_DEV, B * H, 128), jnp.bfloat16),
            pltpu.SemaphoreType.DMA((2, N_DEV - 1)),
            pltpu.SemaphoreType.DMA((2, N_DEV - 1)),
        ],
        compiler_params=pltpu.CompilerParams(collective_id=0),
    )(Q, Kp, Vp, bt, lens2)


# device time: 19592 ns/iter; 1.2164x vs baseline; 1.0080x over previous
import jax
import jax.numpy as jnp
from jax import lax
from jax.experimental import pallas as pl
from jax.experimental.pallas import tpu as pltpu

N_DEV = 4
B, H, D, BS = 16, 16, 64, 16
P = 512 // N_DEV
NT = 128


def kernel(Q, K, V, bt, lens):
    lens2 = lens.reshape(B, 1)
    Kp = jnp.transpose(K, (1, 2, 3, 0))
    Vp = jnp.transpose(V, (1, 2, 3, 0))

    def body(q_ref, k_ref, v_ref, bt_ref, lens_ref, out_ref,
             comm_ref, send_sems, recv_sems):
        my = lax.axis_index("i")

        barrier_sem = pltpu.get_barrier_semaphore()
        for t in range(1, N_DEV):
            pl.semaphore_signal(barrier_sem, inc=1,
                                device_id=(lax.rem(my + t, N_DEV),),
                                device_id_type=pl.DeviceIdType.MESH)

        page0 = my * P
        btv = bt_ref[:, :]
        lensv = lens_ref[:, :]
        g = page0 + lax.broadcasted_iota(jnp.int32, (1, 1, P), 2)
        j = lax.broadcasted_iota(jnp.int32, (1, NT, 1), 1)
        match = (btv[:, :, None] == g) & (j < lensv[:, :, None])
        cnt = jnp.sum(match.astype(jnp.float32), axis=1)
        logcnt = jnp.log(cnt)

        scale = D ** -0.5
        HH = H // 2
        q = q_ref[:, 0, :, :].transpose(1, 0, 2)
        rdmas = []

        for w in range(2):
            h0 = w * HH
            qh = q[h0:h0 + HH]
            qb = jnp.broadcast_to(
                qh[None], (BS, HH, B, D)).reshape(BS * HH, B, D)
            k = k_ref[:, h0:h0 + HH, :, :].reshape(BS * HH, D, P)
            v = v_ref[:, h0:h0 + HH, :, :].reshape(BS * HH, D, P)
            s = jax.lax.dot_general(
                qb, k, (((2,), (1,)), ((0,), (0,))),
                preferred_element_type=jnp.float32) * scale
            s4 = s.reshape(BS, HH, B, P) + logcnt[None, None, :, :]
            m = jnp.maximum(
                jnp.max(s4, axis=(0, 3), keepdims=True), -1e30)
            e4 = jnp.exp(s4 - m)
            l = jnp.sum(e4, axis=(0, 3), keepdims=True)
            o = jax.lax.dot_general(
                e4.reshape(BS * HH, B, P), v, (((2,), (2,)), ((0,), (0,))),
                preferred_element_type=jnp.float32)
            o = jnp.sum(o.reshape(BS, HH, B, D), axis=0)

            r0 = h0 * B
            nr = HH * B
            mr = m.reshape(nr, 1)
            lr = l.reshape(nr, 1)
            m_hi = mr.astype(jnp.bfloat16)
            m_lo = (mr - m_hi.astype(jnp.float32)).astype(jnp.bfloat16)
            l_hi = lr.astype(jnp.bfloat16)
            l_lo = (lr - l_hi.astype(jnp.float32)).astype(jnp.bfloat16)
            comm_ref[0, pl.ds(r0, nr), 0:D] = o.reshape(nr, D).astype(
                jnp.bfloat16)
            comm_ref[0, pl.ds(r0, nr), D:D + 1] = m_hi
            comm_ref[0, pl.ds(r0, nr), D + 1:D + 2] = m_lo
            comm_ref[0, pl.ds(r0, nr), D + 2:D + 3] = l_hi
            comm_ref[0, pl.ds(r0, nr), D + 3:D + 4] = l_lo

            if w == 0:
                pl.semaphore_wait(barrier_sem, N_DEV - 1)

            for t in range(1, N_DEV):
                rdma = pltpu.make_async_remote_copy(
                    src_ref=comm_ref.at[0, pl.ds(r0, nr)],
                    dst_ref=comm_ref.at[N_DEV - t, pl.ds(r0, nr)],
                    send_sem=send_sems.at[w, N_DEV - 1 - t],
                    recv_sem=recv_sems.at[w, N_DEV - 1 - t],
                    device_id=(lax.rem(my + t, N_DEV),),
                    device_id_type=pl.DeviceIdType.MESH,
                )
                rdma.start()
                rdmas.append(rdma)

        for w in range(2):
            for rdma in rdmas[w * 3:(w + 1) * 3]:
                rdma.wait_recv()
            h0 = w * HH
            r0 = h0 * B
            nr = HH * B
            blk = comm_ref[:, pl.ds(r0, nr), :]
            os_ = blk[:, :, 0:D].astype(jnp.float32)
            ms = (blk[:, :, D:D + 1].astype(jnp.float32)
                  + blk[:, :, D + 1:D + 2].astype(jnp.float32))
            ls = (blk[:, :, D + 2:D + 3].astype(jnp.float32)
                  + blk[:, :, D + 3:D + 4].astype(jnp.float32))
            mg = jnp.max(ms, axis=0)
            sc = jnp.exp(ms - mg[None, :, :])
            lg = jnp.sum(ls * sc, axis=0)
            og = jnp.sum(os_ * sc, axis=0)
            out = og / lg
            out_ref[:, :, h0:h0 + HH, :] = (
                out.reshape(HH, B, D).transpose(1, 0, 2).reshape(B, 1, HH, D)
            )

        for rdma in rdmas:
            rdma.wait_send()

    return pl.pallas_call(
        body,
        out_shape=jax.ShapeDtypeStruct((B, 1, H, D), jnp.float32),
        in_specs=[pl.BlockSpec(memory_space=pltpu.VMEM)] * 5,
        out_specs=pl.BlockSpec(memory_space=pltpu.VMEM),
        scratch_shapes=[
            pltpu.VMEM((N_DEV, B * H, 128), jnp.bfloat16),
            pltpu.SemaphoreType.DMA((2, N_DEV - 1)),
            pltpu.SemaphoreType.DMA((2, N_DEV - 1)),
        ],
        compiler_params=pltpu.CompilerParams(collective_id=0),
    )(Q, Kp, Vp, bt, lens2)
